# baseline (speedup 1.0000x reference)
import functools

import jax
import jax.numpy as jnp
from jax import lax
from jax.experimental import pallas as pl
from jax.experimental.pallas import tpu as pltpu
from jax.experimental.pallas import tpu_sc as plsc

N_NODES = 10000
DIM = 256
HIDDEN = 512
N_EDGES = 160000
HALF = DIM // 2
N_SUBCORES = 16
CHUNK = 128
CHUNKS_PER_TILE = 80
EDGES_PER_TILE = CHUNK * CHUNKS_PER_TILE
EDGES_PAD = EDGES_PER_TILE * N_SUBCORES
ROWS_PAD = 10240
ROWS_PER_TILE = ROWS_PAD // N_SUBCORES


NBUF = 2
GRP = 2
N_GROUPS = CHUNKS_PER_TILE // GRP


def _sc_body(xh, srch, dsth, zh, outh, acc, src_v, dst_v, rows_v,
             gs0, gs1, ss0, ss1, is0, is1):
    c = lax.axis_index("c")
    s = lax.axis_index("s")
    base_r = s * ROWS_PER_TILE
    base_c = s * CHUNKS_PER_TILE
    gsem = (gs0, gs1)
    ssem = (ss0, ss1)
    isem = (is0, is1)

    def idx_group_descs(g, slot):
        src_sl = pl.ds(base_c + g * GRP, GRP)
        dst_sl = pl.ds(GRP * slot, GRP)
        return (pltpu.make_async_copy(srch.at[src_sl], src_v.at[dst_sl],
                                      isem[slot]),
                pltpu.make_async_copy(dsth.at[src_sl], dst_v.at[dst_sl],
                                      isem[slot]))

    def fire_idx_group(g, slot):
        a, b = idx_group_descs(g, slot)
        a.start()
        b.start()

    def wait_idx_group(g, slot):
        a, b = idx_group_descs(g, slot)
        a.wait()
        b.wait()

    def gather_desc(col, row, b):
        return pltpu.make_async_copy(
            xh.at[src_v.at[row], pl.ds(col, HALF)], rows_v.at[b], gsem[b])

    def scatter_desc(row, b):
        return pltpu.make_async_copy(rows_v.at[b], acc.at[dst_v.at[row]],
                                     ssem[b])

    def fire_gather(col, row, b):
        pltpu.async_copy(xh.at[src_v.at[row], pl.ds(col, HALF)], rows_v.at[b],
                         gsem[b])

    def fire_scatter(row, b):
        pltpu.async_copy(rows_v.at[b], acc.at[dst_v.at[row]], ssem[b],
                         add=True)

    def chunk_step(g, slot, r, col, first_group=False, fetch=True):
        S = r % 2
        Sp = (r + 1) % 2
        my_row = GRP * slot + r
        prev_row = (GRP * slot + r - 1 if r >= 1
                    else GRP * (1 - slot) + GRP - 1)
        pp_row = (GRP * slot + r - 2 if r >= 2
                  else GRP * (1 - slot) + GRP + r - 2)
        if r == 0:
            wait_idx_group(g, slot)
        if not (first_group and r <= 1):
            scatter_desc(pp_row, S).wait()
        if fetch and r == GRP - 1:
            fire_idx_group(g + 1, 1 - slot)
        fire_gather(col, my_row, S)
        if not (first_group and r == 0):
            gather_desc(col, prev_row, Sp).wait()
            fire_scatter(prev_row, Sp)

    fire_idx_group(0, 0)
    fire_idx_group(1, 1)
    pltpu.sync_copy(zh, rows_v.at[0])
    for j in range(ROWS_PER_TILE // CHUNK):
        pltpu.sync_copy(rows_v.at[0], acc.at[pl.ds(base_r + j * CHUNK, CHUNK)])

    plsc.subcore_barrier()

    def main(col):
        for r in range(GRP):
            chunk_step(0, 0, r, col, first_group=True, fetch=False)

        def body(k, carry):
            g1 = 1 + 2 * k
            for r in range(GRP):
                chunk_step(g1, 1, r, col)
            for r in range(GRP):
                chunk_step(g1 + 1, 0, r, col)
            return carry

        lax.fori_loop(0, (N_GROUPS - 2) // 2, body, 0)

        for r in range(GRP):
            chunk_step(N_GROUPS - 1, 1, r, col, fetch=False)

        gather_desc(col, 2 * GRP - 1, 1).wait()
        fire_scatter(2 * GRP - 1, 1)
        scatter_desc(2 * GRP - 2, 0).wait()
        scatter_desc(2 * GRP - 1, 1).wait()

    @pl.when(c == 0)
    def _():
        main(0)

    @pl.when(c == 1)
    def _():
        main(HALF)

    plsc.subcore_barrier()

    def write_out(col):
        sl = pl.ds(base_r, ROWS_PER_TILE)
        pltpu.sync_copy(acc.at[sl], outh.at[sl, pl.ds(col, HALF)])

    @pl.when(c == 0)
    def _():
        write_out(0)

    @pl.when(c == 1)
    def _():
        write_out(HALF)


@functools.cache
def _sc_agg_fn():
    return pl.kernel(
        _sc_body,
        out_type=jax.ShapeDtypeStruct((ROWS_PAD, DIM), jnp.float32),
        mesh=plsc.VectorSubcoreMesh(core_axis_name="c", subcore_axis_name="s"),
        scratch_types=[
            pltpu.VMEM_SHARED((ROWS_PAD, HALF), jnp.float32),
            pltpu.VMEM((2 * GRP, CHUNK), jnp.int32),
            pltpu.VMEM((2 * GRP, CHUNK), jnp.int32),
            pltpu.VMEM((NBUF, CHUNK, HALF), jnp.float32),
        ] + [pltpu.SemaphoreType.DMA] * 6,
    )


MLP_NB = 5
MLP_BLK = N_NODES // MLP_NB
_INV_N = 1.0 / N_NODES


def _mlp_body(x_ref, a_ref, w1_ref, b1_ref, g1_ref, be1_ref, w2_ref,
              b2_ref, g2_ref, be2_ref, o_ref, h_ref, h2_ref, s1_ref, q1_ref,
              s2_ref, q2_ref, m1_ref, sc1_ref, m2_ref, sc2_ref):
    p = pl.program_id(0)
    i = pl.program_id(1)

    @pl.when(p == 0)
    def _():
        @pl.when(i == 0)
        def _():
            s1_ref[...] = jnp.zeros_like(s1_ref)
            q1_ref[...] = jnp.zeros_like(q1_ref)

        xa = x_ref[...] + a_ref[...]
        h = jnp.dot(xa.astype(jnp.bfloat16), w1_ref[...],
                    preferred_element_type=jnp.float32) + b1_ref[...]
        h_ref[pl.ds(i * MLP_BLK, MLP_BLK), :] = h
        s1_ref[...] += jnp.sum(h, axis=0, keepdims=True)
        q1_ref[...] += jnp.sum(h * h, axis=0, keepdims=True)

    @pl.when(p == 1)
    def _():
        @pl.when(i == 0)
        def _():
            mean = s1_ref[...] * _INV_N
            var = q1_ref[...] * _INV_N - mean * mean
            m1_ref[...] = mean
            sc1_ref[...] = lax.rsqrt(var + 1e-5) * g1_ref[...]
            s2_ref[...] = jnp.zeros_like(s2_ref)
            q2_ref[...] = jnp.zeros_like(q2_ref)

        h = h_ref[pl.ds(i * MLP_BLK, MLP_BLK), :]
        hn = (h - m1_ref[...]) * sc1_ref[...] + be1_ref[...]
        hb = jnp.maximum(hn, 0.0).astype(jnp.bfloat16)
        h2 = jnp.dot(hb, w2_ref[...],
                     preferred_element_type=jnp.float32) + b2_ref[...]
        h2_ref[pl.ds(i * MLP_BLK, MLP_BLK), :] = h2
        s2_ref[...] += jnp.sum(h2, axis=0, keepdims=True)
        q2_ref[...] += jnp.sum(h2 * h2, axis=0, keepdims=True)

    @pl.when(p == 2)
    def _():
        @pl.when(i == 0)
        def _():
            mean2 = s2_ref[...] * _INV_N
            var2 = q2_ref[...] * _INV_N - mean2 * mean2
            m2_ref[...] = mean2
            sc2_ref[...] = lax.rsqrt(var2 + 1e-5) * g2_ref[...]

        h2 = h2_ref[pl.ds(i * MLP_BLK, MLP_BLK), :]
        hn2 = (h2 - m2_ref[...]) * sc2_ref[...] + be2_ref[...]
        o_ref[...] = x_ref[...] + jnp.maximum(hn2, 0.0)


def _mlp(x, a, W1, b1, g1, be1, W2, b2, g2, be2, interpret=False):
    def xmap(p, i):
        return (jnp.where(p == 1, 0, i), 0)

    def amap(p, i):
        return (jnp.where(p == 0, i, 0), 0)

    def omap(p, i):
        return (jnp.where(p == 2, i, 0), 0)

    full = lambda p, i: (0, 0)
    return pl.pallas_call(
        _mlp_body,
        grid=(3, MLP_NB),
        out_shape=jax.ShapeDtypeStruct((N_NODES, DIM), jnp.float32),
        in_specs=[
            pl.BlockSpec((MLP_BLK, DIM), xmap),
            pl.BlockSpec((MLP_BLK, DIM), amap),
            pl.BlockSpec((DIM, HIDDEN), full),
            pl.BlockSpec((1, HIDDEN), full),
            pl.BlockSpec((1, HIDDEN), full),
            pl.BlockSpec((1, HIDDEN), full),
            pl.BlockSpec((HIDDEN, DIM), full),
            pl.BlockSpec((1, DIM), full),
            pl.BlockSpec((1, DIM), full),
            pl.BlockSpec((1, DIM), full),
        ],
        out_specs=pl.BlockSpec((MLP_BLK, DIM), omap),
        scratch_shapes=[
            pltpu.VMEM((N_NODES, HIDDEN), jnp.float32),
            pltpu.VMEM((N_NODES, DIM), jnp.float32),
            pltpu.VMEM((1, HIDDEN), jnp.float32),
            pltpu.VMEM((1, HIDDEN), jnp.float32),
            pltpu.VMEM((1, DIM), jnp.float32),
            pltpu.VMEM((1, DIM), jnp.float32),
            pltpu.VMEM((1, HIDDEN), jnp.float32),
            pltpu.VMEM((1, HIDDEN), jnp.float32),
            pltpu.VMEM((1, DIM), jnp.float32),
            pltpu.VMEM((1, DIM), jnp.float32),
        ],
        interpret=interpret,
    )(x, a, W1.astype(jnp.bfloat16), b1.reshape(1, HIDDEN),
      g1.reshape(1, HIDDEN), be1.reshape(1, HIDDEN),
      W2.astype(jnp.bfloat16), b2.reshape(1, DIM), g2.reshape(1, DIM),
      be2.reshape(1, DIM))


def kernel(x, edge_index, W1, b1, g1, be1, W2, b2, g2, be2):
    src = edge_index[0].astype(jnp.int32)
    dst = edge_index[1].astype(jnp.int32)
    pad = EDGES_PAD - N_EDGES
    pad_iota = jnp.arange(pad, dtype=jnp.int32)
    srcp = jnp.concatenate([src, pad_iota % N_NODES]).reshape(-1, CHUNK)
    dstp = jnp.concatenate(
        [dst, N_NODES + pad_iota % (ROWS_PAD - N_NODES)]).reshape(-1, CHUNK)
    zeros_blk = jnp.zeros((CHUNK, HALF), jnp.float32)
    a = _sc_agg_fn()(x, srcp, dstp, zeros_blk)
    return _mlp(x, a, W1, b1, g1, be1, W2, b2, g2, be2)


# speedup vs baseline: 1.0085x; 1.0085x over previous
import functools

import jax
import jax.numpy as jnp
from jax import lax
from jax.experimental import pallas as pl
from jax.experimental.pallas import tpu as pltpu
from jax.experimental.pallas import tpu_sc as plsc

N_NODES = 10000
DIM = 256
HIDDEN = 512
N_EDGES = 160000
HALF = DIM // 2
N_SUBCORES = 16
CHUNK = 128
CHUNKS_PER_TILE = 80
EDGES_PER_TILE = CHUNK * CHUNKS_PER_TILE
EDGES_PAD = EDGES_PER_TILE * N_SUBCORES
ROWS_PAD = 10240
ROWS_PER_TILE = ROWS_PAD // N_SUBCORES


NBUF = 2
GRP = 2
N_GROUPS = CHUNKS_PER_TILE // GRP


def _sc_body(xh, srch, dsth, zh, outh, acc, src_v, dst_v, rows_v,
             gs0, gs1, ss0, ss1, is0, is1):
    c = lax.axis_index("c")
    s = lax.axis_index("s")
    base_r = s * ROWS_PER_TILE
    base_c = s * CHUNKS_PER_TILE
    gsem = (gs0, gs1)
    ssem = (ss0, ss1)
    isem = (is0, is1)

    def idx_group_descs(g, slot):
        src_sl = pl.ds(base_c + g * GRP, GRP)
        dst_sl = pl.ds(GRP * slot, GRP)
        return (pltpu.make_async_copy(srch.at[src_sl], src_v.at[dst_sl],
                                      isem[slot]),
                pltpu.make_async_copy(dsth.at[src_sl], dst_v.at[dst_sl],
                                      isem[slot]))

    def fire_idx_group(g, slot):
        a, b = idx_group_descs(g, slot)
        a.start()
        b.start()

    def wait_idx_group(g, slot):
        a, b = idx_group_descs(g, slot)
        a.wait()
        b.wait()

    def gather_desc(col, row, b):
        return pltpu.make_async_copy(
            xh.at[src_v.at[row], pl.ds(col, HALF)], rows_v.at[b], gsem[b])

    def scatter_desc(row, b):
        return pltpu.make_async_copy(rows_v.at[b], acc.at[dst_v.at[row]],
                                     ssem[b])

    def fire_gather(col, row, b):
        pltpu.async_copy(xh.at[src_v.at[row], pl.ds(col, HALF)], rows_v.at[b],
                         gsem[b])

    def fire_scatter(row, b):
        pltpu.async_copy(rows_v.at[b], acc.at[dst_v.at[row]], ssem[b],
                         add=True)

    def chunk_step(g, slot, r, col, first_group=False, fetch=True):
        S = r % 2
        Sp = (r + 1) % 2
        my_row = GRP * slot + r
        prev_row = (GRP * slot + r - 1 if r >= 1
                    else GRP * (1 - slot) + GRP - 1)
        pp_row = (GRP * slot + r - 2 if r >= 2
                  else GRP * (1 - slot) + GRP + r - 2)
        if r == 0:
            wait_idx_group(g, slot)
        if not (first_group and r <= 1):
            scatter_desc(pp_row, S).wait()
        if fetch and r == GRP - 1:
            fire_idx_group(g + 1, 1 - slot)
        fire_gather(col, my_row, S)
        if not (first_group and r == 0):
            gather_desc(col, prev_row, Sp).wait()
            fire_scatter(prev_row, Sp)

    fire_idx_group(0, 0)
    fire_idx_group(1, 1)
    pltpu.sync_copy(zh, rows_v.at[0])
    for j in range(ROWS_PER_TILE // CHUNK):
        pltpu.sync_copy(rows_v.at[0], acc.at[pl.ds(base_r + j * CHUNK, CHUNK)])

    plsc.subcore_barrier()

    def main(col):
        for r in range(GRP):
            chunk_step(0, 0, r, col, first_group=True, fetch=False)

        def body(k, carry):
            g1 = 1 + 2 * k
            for r in range(GRP):
                chunk_step(g1, 1, r, col)
            for r in range(GRP):
                chunk_step(g1 + 1, 0, r, col)
            return carry

        lax.fori_loop(0, (N_GROUPS - 2) // 2, body, 0)

        for r in range(GRP):
            chunk_step(N_GROUPS - 1, 1, r, col, fetch=False)

        gather_desc(col, 2 * GRP - 1, 1).wait()
        fire_scatter(2 * GRP - 1, 1)
        scatter_desc(2 * GRP - 2, 0).wait()
        scatter_desc(2 * GRP - 1, 1).wait()

    @pl.when(c == 0)
    def _():
        main(0)

    @pl.when(c == 1)
    def _():
        main(HALF)

    plsc.subcore_barrier()

    def write_out(col):
        nblk = ROWS_PER_TILE // CHUNK

        def load(j, b):
            sl = pl.ds(base_r + j * CHUNK, CHUNK)
            return pltpu.make_async_copy(acc.at[sl], rows_v.at[b], gsem[b])

        def store(j, b):
            sl = pl.ds(base_r + j * CHUNK, CHUNK)
            return pltpu.make_async_copy(
                rows_v.at[b], outh.at[sl, pl.ds(col, HALF)], ssem[b])

        load(0, 0).start()
        for j in range(nblk):
            b = j % NBUF
            bn = (j + 1) % NBUF
            load(j, b).wait()
            store(j, b).start()
            if j + 1 < nblk:
                if j + 1 >= NBUF:
                    store(j + 1 - NBUF, bn).wait()
                load(j + 1, bn).start()
        store(nblk - 2, (nblk - 2) % NBUF).wait()
        store(nblk - 1, (nblk - 1) % NBUF).wait()

    @pl.when(c == 0)
    def _():
        write_out(0)

    @pl.when(c == 1)
    def _():
        write_out(HALF)


@functools.cache
def _sc_agg_fn():
    return pl.kernel(
        _sc_body,
        out_type=jax.ShapeDtypeStruct((ROWS_PAD, DIM), jnp.float32),
        mesh=plsc.VectorSubcoreMesh(core_axis_name="c", subcore_axis_name="s"),
        scratch_types=[
            pltpu.VMEM_SHARED((ROWS_PAD, HALF), jnp.float32),
            pltpu.VMEM((2 * GRP, CHUNK), jnp.int32),
            pltpu.VMEM((2 * GRP, CHUNK), jnp.int32),
            pltpu.VMEM((NBUF, CHUNK, HALF), jnp.float32),
        ] + [pltpu.SemaphoreType.DMA] * 6,
    )


MLP_NB = 5
MLP_BLK = N_NODES // MLP_NB
_INV_N = 1.0 / N_NODES


def _mlp_body(x_ref, a_ref, w1_ref, b1_ref, g1_ref, be1_ref, w2_ref,
              b2_ref, g2_ref, be2_ref, o_ref, h_ref, h2_ref, s1_ref, q1_ref,
              s2_ref, q2_ref, m1_ref, sc1_ref, m2_ref, sc2_ref):
    p = pl.program_id(0)
    i = pl.program_id(1)

    @pl.when(p == 0)
    def _():
        @pl.when(i == 0)
        def _():
            s1_ref[...] = jnp.zeros_like(s1_ref)
            q1_ref[...] = jnp.zeros_like(q1_ref)

        xa = x_ref[...] + a_ref[...]
        h = jnp.dot(xa.astype(jnp.bfloat16), w1_ref[...],
                    preferred_element_type=jnp.float32) + b1_ref[...]
        h_ref[pl.ds(i * MLP_BLK, MLP_BLK), :] = h
        s1_ref[...] += jnp.sum(h, axis=0, keepdims=True)
        q1_ref[...] += jnp.sum(h * h, axis=0, keepdims=True)

    @pl.when(p == 1)
    def _():
        @pl.when(i == 0)
        def _():
            mean = s1_ref[...] * _INV_N
            var = q1_ref[...] * _INV_N - mean * mean
            m1_ref[...] = mean
            sc1_ref[...] = lax.rsqrt(var + 1e-5) * g1_ref[...]
            s2_ref[...] = jnp.zeros_like(s2_ref)
            q2_ref[...] = jnp.zeros_like(q2_ref)

        h = h_ref[pl.ds(i * MLP_BLK, MLP_BLK), :]
        hn = (h - m1_ref[...]) * sc1_ref[...] + be1_ref[...]
        hb = jnp.maximum(hn, 0.0).astype(jnp.bfloat16)
        h2 = jnp.dot(hb, w2_ref[...],
                     preferred_element_type=jnp.float32) + b2_ref[...]
        h2_ref[pl.ds(i * MLP_BLK, MLP_BLK), :] = h2
        s2_ref[...] += jnp.sum(h2, axis=0, keepdims=True)
        q2_ref[...] += jnp.sum(h2 * h2, axis=0, keepdims=True)

    @pl.when(p == 2)
    def _():
        @pl.when(i == 0)
        def _():
            mean2 = s2_ref[...] * _INV_N
            var2 = q2_ref[...] * _INV_N - mean2 * mean2
            m2_ref[...] = mean2
            sc2_ref[...] = lax.rsqrt(var2 + 1e-5) * g2_ref[...]

        h2 = h2_ref[pl.ds(i * MLP_BLK, MLP_BLK), :]
        hn2 = (h2 - m2_ref[...]) * sc2_ref[...] + be2_ref[...]
        o_ref[...] = x_ref[...] + jnp.maximum(hn2, 0.0)


def _mlp(x, a, W1, b1, g1, be1, W2, b2, g2, be2, interpret=False):
    def xmap(p, i):
        return (jnp.where(p == 1, 0, i), 0)

    def amap(p, i):
        return (jnp.where(p == 0, i, 0), 0)

    def omap(p, i):
        return (jnp.where(p == 2, i, 0), 0)

    full = lambda p, i: (0, 0)
    return pl.pallas_call(
        _mlp_body,
        grid=(3, MLP_NB),
        out_shape=jax.ShapeDtypeStruct((N_NODES, DIM), jnp.float32),
        in_specs=[
            pl.BlockSpec((MLP_BLK, DIM), xmap),
            pl.BlockSpec((MLP_BLK, DIM), amap),
            pl.BlockSpec((DIM, HIDDEN), full),
            pl.BlockSpec((1, HIDDEN), full),
            pl.BlockSpec((1, HIDDEN), full),
            pl.BlockSpec((1, HIDDEN), full),
            pl.BlockSpec((HIDDEN, DIM), full),
            pl.BlockSpec((1, DIM), full),
            pl.BlockSpec((1, DIM), full),
            pl.BlockSpec((1, DIM), full),
        ],
        out_specs=pl.BlockSpec((MLP_BLK, DIM), omap),
        scratch_shapes=[
            pltpu.VMEM((N_NODES, HIDDEN), jnp.float32),
            pltpu.VMEM((N_NODES, DIM), jnp.float32),
            pltpu.VMEM((1, HIDDEN), jnp.float32),
            pltpu.VMEM((1, HIDDEN), jnp.float32),
            pltpu.VMEM((1, DIM), jnp.float32),
            pltpu.VMEM((1, DIM), jnp.float32),
            pltpu.VMEM((1, HIDDEN), jnp.float32),
            pltpu.VMEM((1, HIDDEN), jnp.float32),
            pltpu.VMEM((1, DIM), jnp.float32),
            pltpu.VMEM((1, DIM), jnp.float32),
        ],
        interpret=interpret,
    )(x, a, W1.astype(jnp.bfloat16), b1.reshape(1, HIDDEN),
      g1.reshape(1, HIDDEN), be1.reshape(1, HIDDEN),
      W2.astype(jnp.bfloat16), b2.reshape(1, DIM), g2.reshape(1, DIM),
      be2.reshape(1, DIM))


def kernel(x, edge_index, W1, b1, g1, be1, W2, b2, g2, be2):
    src = edge_index[0].astype(jnp.int32)
    dst = edge_index[1].astype(jnp.int32)
    pad = EDGES_PAD - N_EDGES
    pad_iota = jnp.arange(pad, dtype=jnp.int32)
    srcp = jnp.concatenate([src, pad_iota % N_NODES]).reshape(-1, CHUNK)
    dstp = jnp.concatenate(
        [dst, N_NODES + pad_iota % (ROWS_PAD - N_NODES)]).reshape(-1, CHUNK)
    zeros_blk = jnp.zeros((CHUNK, HALF), jnp.float32)
    a = _sc_agg_fn()(x, srcp, dstp, zeros_blk)
    return _mlp(x, a, W1, b1, g1, be1, W2, b2, g2, be2)


# speedup vs baseline: 1.0143x; 1.0057x over previous
import functools

import jax
import jax.numpy as jnp
from jax import lax
from jax.experimental import pallas as pl
from jax.experimental.pallas import tpu as pltpu
from jax.experimental.pallas import tpu_sc as plsc

N_NODES = 10000
DIM = 256
HIDDEN = 512
N_EDGES = 160000
HALF = DIM // 2
N_SUBCORES = 16
CHUNK = 128
CHUNKS_PER_TILE = 80
EDGES_PER_TILE = CHUNK * CHUNKS_PER_TILE
EDGES_PAD = EDGES_PER_TILE * N_SUBCORES
ROWS_PAD = 10240
ROWS_PER_TILE = ROWS_PAD // N_SUBCORES


NBUF = 2
GRP = 2
N_GROUPS = CHUNKS_PER_TILE // GRP


def _sc_body(xh, srch, dsth, zh, outh, acc, src_v, dst_v, rows_v,
             gs0, gs1, ss0, ss1, is0, is1):
    c = lax.axis_index("c")
    s = lax.axis_index("s")
    base_r = s * ROWS_PER_TILE
    base_c = s * CHUNKS_PER_TILE
    gsem = (gs0, gs1)
    ssem = (ss0, ss1)
    isem = (is0, is1)

    def idx_group_descs(g, slot):
        src_sl = pl.ds(base_c + g * GRP, GRP)
        dst_sl = pl.ds(GRP * slot, GRP)
        return (pltpu.make_async_copy(srch.at[src_sl], src_v.at[dst_sl],
                                      isem[slot]),
                pltpu.make_async_copy(dsth.at[src_sl], dst_v.at[dst_sl],
                                      isem[slot]))

    def fire_idx_group(g, slot):
        a, b = idx_group_descs(g, slot)
        a.start()
        b.start()

    def wait_idx_group(g, slot):
        a, b = idx_group_descs(g, slot)
        a.wait()
        b.wait()

    def gather_desc(col, row, b):
        return pltpu.make_async_copy(
            xh.at[src_v.at[row], pl.ds(col, HALF)], rows_v.at[b], gsem[b])

    def scatter_desc(row, b):
        return pltpu.make_async_copy(rows_v.at[b], acc.at[dst_v.at[row]],
                                     ssem[b])

    def fire_gather(col, row, b):
        pltpu.async_copy(xh.at[src_v.at[row], pl.ds(col, HALF)], rows_v.at[b],
                         gsem[b])

    def fire_scatter(row, b):
        pltpu.async_copy(rows_v.at[b], acc.at[dst_v.at[row]], ssem[b],
                         add=True)

    def chunk_step(g, slot, r, col, first_group=False, fetch=True):
        S = r % 2
        Sp = (r + 1) % 2
        my_row = GRP * slot + r
        prev_row = (GRP * slot + r - 1 if r >= 1
                    else GRP * (1 - slot) + GRP - 1)
        pp_row = (GRP * slot + r - 2 if r >= 2
                  else GRP * (1 - slot) + GRP + r - 2)
        if r == 0:
            wait_idx_group(g, slot)
        if not (first_group and r <= 1):
            scatter_desc(pp_row, S).wait()
        if fetch and r == GRP - 1:
            fire_idx_group(g + 1, 1 - slot)
        fire_gather(col, my_row, S)
        if not (first_group and r == 0):
            gather_desc(col, prev_row, Sp).wait()
            fire_scatter(prev_row, Sp)

    fire_idx_group(0, 0)
    fire_idx_group(1, 1)
    pltpu.sync_copy(zh, rows_v.at[0])
    nblk0 = ROWS_PER_TILE // CHUNK

    def zero_desc(j):
        return pltpu.make_async_copy(
            rows_v.at[0], acc.at[pl.ds(base_r + j * CHUNK, CHUNK)], ssem[0])

    for j in range(nblk0):
        zero_desc(j).start()
    for j in range(nblk0):
        zero_desc(j).wait()

    def prologue(col):
        wait_idx_group(0, 0)
        fire_gather(col, 0, 0)
        fire_gather(col, 1, 1)

    @pl.when(c == 0)
    def _():
        prologue(0)

    @pl.when(c == 1)
    def _():
        prologue(HALF)

    plsc.subcore_barrier()

    def main(col):
        gather_desc(col, 0, 0).wait()
        fire_scatter(0, 0)

        def body(k, carry):
            g1 = 1 + 2 * k
            for r in range(GRP):
                chunk_step(g1, 1, r, col)
            for r in range(GRP):
                chunk_step(g1 + 1, 0, r, col)
            return carry

        lax.fori_loop(0, (N_GROUPS - 2) // 2, body, 0)

        for r in range(GRP):
            chunk_step(N_GROUPS - 1, 1, r, col, fetch=False)

        gather_desc(col, 2 * GRP - 1, 1).wait()
        fire_scatter(2 * GRP - 1, 1)
        scatter_desc(2 * GRP - 2, 0).wait()
        scatter_desc(2 * GRP - 1, 1).wait()

    @pl.when(c == 0)
    def _():
        main(0)

    @pl.when(c == 1)
    def _():
        main(HALF)

    plsc.subcore_barrier()

    def write_out(col):
        nblk = ROWS_PER_TILE // CHUNK

        def load(j, b):
            sl = pl.ds(base_r + j * CHUNK, CHUNK)
            return pltpu.make_async_copy(acc.at[sl], rows_v.at[b], gsem[b])

        def store(j, b):
            sl = pl.ds(base_r + j * CHUNK, CHUNK)
            return pltpu.make_async_copy(
                rows_v.at[b], outh.at[sl, pl.ds(col, HALF)], ssem[b])

        load(0, 0).start()
        for j in range(nblk):
            b = j % NBUF
            bn = (j + 1) % NBUF
            load(j, b).wait()
            store(j, b).start()
            if j + 1 < nblk:
                if j + 1 >= NBUF:
                    store(j + 1 - NBUF, bn).wait()
                load(j + 1, bn).start()
        store(nblk - 2, (nblk - 2) % NBUF).wait()
        store(nblk - 1, (nblk - 1) % NBUF).wait()

    @pl.when(c == 0)
    def _():
        write_out(0)

    @pl.when(c == 1)
    def _():
        write_out(HALF)


@functools.cache
def _sc_agg_fn():
    return pl.kernel(
        _sc_body,
        out_type=jax.ShapeDtypeStruct((ROWS_PAD, DIM), jnp.float32),
        mesh=plsc.VectorSubcoreMesh(core_axis_name="c", subcore_axis_name="s"),
        scratch_types=[
            pltpu.VMEM_SHARED((ROWS_PAD, HALF), jnp.float32),
            pltpu.VMEM((2 * GRP, CHUNK), jnp.int32),
            pltpu.VMEM((2 * GRP, CHUNK), jnp.int32),
            pltpu.VMEM((NBUF, CHUNK, HALF), jnp.float32),
        ] + [pltpu.SemaphoreType.DMA] * 6,
    )


MLP_NB = 5
MLP_BLK = N_NODES // MLP_NB
_INV_N = 1.0 / N_NODES


def _mlp_body(x_ref, a_ref, w1_ref, b1_ref, g1_ref, be1_ref, w2_ref,
              b2_ref, g2_ref, be2_ref, o_ref, h_ref, h2_ref, s1_ref, q1_ref,
              s2_ref, q2_ref, m1_ref, sc1_ref, m2_ref, sc2_ref):
    p = pl.program_id(0)
    i = pl.program_id(1)

    @pl.when(p == 0)
    def _():
        @pl.when(i == 0)
        def _():
            s1_ref[...] = jnp.zeros_like(s1_ref)
            q1_ref[...] = jnp.zeros_like(q1_ref)

        xa = x_ref[...] + a_ref[...]
        h = jnp.dot(xa.astype(jnp.bfloat16), w1_ref[...],
                    preferred_element_type=jnp.float32) + b1_ref[...]
        h_ref[pl.ds(i * MLP_BLK, MLP_BLK), :] = h
        s1_ref[...] += jnp.sum(h, axis=0, keepdims=True)
        q1_ref[...] += jnp.sum(h * h, axis=0, keepdims=True)

    @pl.when(p == 1)
    def _():
        @pl.when(i == 0)
        def _():
            mean = s1_ref[...] * _INV_N
            var = q1_ref[...] * _INV_N - mean * mean
            m1_ref[...] = mean
            sc1_ref[...] = lax.rsqrt(var + 1e-5) * g1_ref[...]
            s2_ref[...] = jnp.zeros_like(s2_ref)
            q2_ref[...] = jnp.zeros_like(q2_ref)

        h = h_ref[pl.ds(i * MLP_BLK, MLP_BLK), :]
        hn = (h - m1_ref[...]) * sc1_ref[...] + be1_ref[...]
        hb = jnp.maximum(hn, 0.0).astype(jnp.bfloat16)
        h2 = jnp.dot(hb, w2_ref[...],
                     preferred_element_type=jnp.float32) + b2_ref[...]
        h2_ref[pl.ds(i * MLP_BLK, MLP_BLK), :] = h2
        s2_ref[...] += jnp.sum(h2, axis=0, keepdims=True)
        q2_ref[...] += jnp.sum(h2 * h2, axis=0, keepdims=True)

    @pl.when(p == 2)
    def _():
        @pl.when(i == 0)
        def _():
            mean2 = s2_ref[...] * _INV_N
            var2 = q2_ref[...] * _INV_N - mean2 * mean2
            m2_ref[...] = mean2
            sc2_ref[...] = lax.rsqrt(var2 + 1e-5) * g2_ref[...]

        h2 = h2_ref[pl.ds(i * MLP_BLK, MLP_BLK), :]
        hn2 = (h2 - m2_ref[...]) * sc2_ref[...] + be2_ref[...]
        o_ref[...] = x_ref[...] + jnp.maximum(hn2, 0.0)


def _mlp(x, a, W1, b1, g1, be1, W2, b2, g2, be2, interpret=False):
    def xmap(p, i):
        return (jnp.where(p == 1, 0, i), 0)

    def amap(p, i):
        return (jnp.where(p == 0, i, 0), 0)

    def omap(p, i):
        return (jnp.where(p == 2, i, 0), 0)

    full = lambda p, i: (0, 0)
    return pl.pallas_call(
        _mlp_body,
        grid=(3, MLP_NB),
        out_shape=jax.ShapeDtypeStruct((N_NODES, DIM), jnp.float32),
        in_specs=[
            pl.BlockSpec((MLP_BLK, DIM), xmap),
            pl.BlockSpec((MLP_BLK, DIM), amap),
            pl.BlockSpec((DIM, HIDDEN), full),
            pl.BlockSpec((1, HIDDEN), full),
            pl.BlockSpec((1, HIDDEN), full),
            pl.BlockSpec((1, HIDDEN), full),
            pl.BlockSpec((HIDDEN, DIM), full),
            pl.BlockSpec((1, DIM), full),
            pl.BlockSpec((1, DIM), full),
            pl.BlockSpec((1, DIM), full),
        ],
        out_specs=pl.BlockSpec((MLP_BLK, DIM), omap),
        scratch_shapes=[
            pltpu.VMEM((N_NODES, HIDDEN), jnp.float32),
            pltpu.VMEM((N_NODES, DIM), jnp.float32),
            pltpu.VMEM((1, HIDDEN), jnp.float32),
            pltpu.VMEM((1, HIDDEN), jnp.float32),
            pltpu.VMEM((1, DIM), jnp.float32),
            pltpu.VMEM((1, DIM), jnp.float32),
            pltpu.VMEM((1, HIDDEN), jnp.float32),
            pltpu.VMEM((1, HIDDEN), jnp.float32),
            pltpu.VMEM((1, DIM), jnp.float32),
            pltpu.VMEM((1, DIM), jnp.float32),
        ],
        interpret=interpret,
    )(x, a, W1.astype(jnp.bfloat16), b1.reshape(1, HIDDEN),
      g1.reshape(1, HIDDEN), be1.reshape(1, HIDDEN),
      W2.astype(jnp.bfloat16), b2.reshape(1, DIM), g2.reshape(1, DIM),
      be2.reshape(1, DIM))


def kernel(x, edge_index, W1, b1, g1, be1, W2, b2, g2, be2):
    src = edge_index[0].astype(jnp.int32)
    dst = edge_index[1].astype(jnp.int32)
    pad = EDGES_PAD - N_EDGES
    pad_iota = jnp.arange(pad, dtype=jnp.int32)
    srcp = jnp.concatenate([src, pad_iota % N_NODES]).reshape(-1, CHUNK)
    dstp = jnp.concatenate(
        [dst, N_NODES + pad_iota % (ROWS_PAD - N_NODES)]).reshape(-1, CHUNK)
    zeros_blk = jnp.zeros((CHUNK, HALF), jnp.float32)
    a = _sc_agg_fn()(x, srcp, dstp, zeros_blk)
    return _mlp(x, a, W1, b1, g1, be1, W2, b2, g2, be2)
